# baseline (speedup 1.0000x reference)
import functools

import jax
import jax.numpy as jnp
from jax import lax
from jax.experimental import pallas as pl
from jax.experimental.pallas import tpu as pltpu
from jax.experimental.pallas import tpu_sc as plsc

N = 10000
D = 128
H = 128
E = 320000
G = 16
OUT = 64

N_PAD = 10240
S3 = 3 * N_PAD
BLK = 1280
N_BLOCKS = S3 // BLK

NW = 32
CH = 80
E_PAD = NW * CH * 128
RPT = N_PAD // 16

DEG_CH = 3 * CH
DEG_PT = S3 // 16


@functools.cache
def _get_deg_call():
    mesh = plsc.VectorSubcoreMesh(core_axis_name="c", subcore_axis_name="s")

    @functools.partial(
        pl.kernel,
        mesh=mesh,
        out_type=jax.ShapeDtypeStruct((2, S3), jnp.float32),
        scratch_types=[
            pltpu.VMEM((DEG_CH, 128), jnp.int32),
            pltpu.VMEM((128,), jnp.float32),
            pltpu.VMEM_SHARED((S3,), jnp.float32),
        ],
    )
    def deg_call(dst_hbm, ones_hbm, zeros_hbm, out_hbm, didx, ones_v, acc):
        cid = lax.axis_index("c")
        sid = lax.axis_index("s")
        w = cid * 16 + sid
        z0 = sid * DEG_PT
        pltpu.sync_copy(zeros_hbm.at[pl.ds(z0, DEG_PT)], acc.at[pl.ds(z0, DEG_PT)])
        pltpu.sync_copy(ones_hbm, ones_v)
        pltpu.sync_copy(dst_hbm.at[w], didx)
        plsc.subcore_barrier()

        def body(j, carry):
            pltpu.sync_copy(ones_v, acc.at[didx.at[j]], add=True)
            return carry

        lax.fori_loop(0, DEG_CH, body, 0)
        plsc.subcore_barrier()
        pltpu.sync_copy(acc.at[pl.ds(z0, DEG_PT)], out_hbm.at[cid].at[pl.ds(z0, DEG_PT)])

    return deg_call


def _deg_call(dst_hbm, ones_hbm, zeros_hbm):
    return _get_deg_call()(dst_hbm, ones_hbm, zeros_hbm)


@functools.cache
def _get_edge_agg():
    mesh = plsc.VectorSubcoreMesh(core_axis_name="c", subcore_axis_name="s")

    @functools.partial(
        pl.kernel,
        mesh=mesh,
        out_type=jax.ShapeDtypeStruct((2, S3, 128), jnp.float32),
        scratch_types=[
            pltpu.VMEM((3, 128), jnp.int32),
            pltpu.VMEM((3, 128), jnp.int32),
            pltpu.VMEM((2, 128, 128), jnp.float32),
            pltpu.VMEM_SHARED((N_PAD, 128), jnp.float32),
            pltpu.SemaphoreType.DMA,
            pltpu.SemaphoreType.DMA,
            pltpu.SemaphoreType.DMA,
        ],
    )
    def edge_agg(h_hbm, src_hbm, dst_hbm, zeros_hbm, out_hbm,
                 sidx, didx, rows, acc, isem, gsem, ssem):
        cid = lax.axis_index("c")
        sid = lax.axis_index("s")
        w = cid * 16 + sid
        r0 = sid * RPT

        def make_phase(b):
            src_w = src_hbm.at[b].at[w]
            dst_w = dst_hbm.at[b].at[w]
            c0 = b * CH

            def fire_i(c):
                pltpu.async_copy(src_w.at[c - c0], sidx.at[(c - c0) % 3], isem)
                pltpu.async_copy(dst_w.at[c - c0], didx.at[(c - c0) % 3], isem)

            def wait_i(c):
                pltpu.make_async_copy(src_w.at[c - c0],
                                      sidx.at[(c - c0) % 3], isem).wait()
                pltpu.make_async_copy(dst_w.at[c - c0],
                                      didx.at[(c - c0) % 3], isem).wait()

            def fire_g(c):
                pltpu.async_copy(h_hbm.at[sidx.at[(c - c0) % 3]],
                                 rows.at[(c - c0) % 2], gsem)

            def wait_g(c):
                pltpu.make_async_copy(h_hbm.at[sidx.at[(c - c0) % 3]],
                                      rows.at[(c - c0) % 2], gsem).wait()

            def fire_s(c):
                pltpu.async_copy(rows.at[(c - c0) % 2],
                                 acc.at[didx.at[(c - c0) % 3]], ssem, add=True)

            def wait_s(c):
                pltpu.make_async_copy(rows.at[(c - c0) % 2],
                                      acc.at[didx.at[(c - c0) % 3]], ssem).wait()

            return c0, fire_i, wait_i, fire_g, wait_g, fire_s, wait_s

        for b in range(3):
            c0, fire_i, wait_i, fire_g, wait_g, fire_s, wait_s = make_phase(b)
            @pl.when(cid == 0)
            def _():
                pltpu.sync_copy(h_hbm.at[pl.ds(b * N_PAD + r0, RPT)],
                                acc.at[pl.ds(r0, RPT)])

            @pl.when(cid != 0)
            def _():
                pltpu.sync_copy(zeros_hbm.at[pl.ds(r0, RPT)],
                                acc.at[pl.ds(r0, RPT)])

            fire_i(c0)
            fire_i(c0 + 1)
            wait_i(c0)
            fire_g(c0)
            plsc.subcore_barrier()

            wait_i(c0 + 1)
            wait_g(c0)
            fire_g(c0 + 1)
            fire_s(c0)
            fire_i(c0 + 2)

            def body(t, carry):
                c = c0 + t
                wait_i(c + 1)
                wait_g(c)
                wait_s(c - 1)
                fire_g(c + 1)
                fire_s(c)
                fire_i(c + 2)
                return carry

            lax.fori_loop(1, CH - 2, body, 0)

            c = c0 + CH - 2
            wait_i(c + 1)
            wait_g(c)
            wait_s(c - 1)
            fire_g(c + 1)
            fire_s(c)
            wait_g(c + 1)
            wait_s(c)
            fire_s(c + 1)
            wait_s(c + 1)

            plsc.subcore_barrier()
            pltpu.sync_copy(acc.at[pl.ds(r0, RPT)],
                            out_hbm.at[cid].at[pl.ds(b * N_PAD + r0, RPT)])
            plsc.subcore_barrier()

    return edge_agg


def _edge_agg(h_hbm, src_hbm, dst_hbm, zeros_hbm):
    return _get_edge_agg()(h_hbm, src_hbm, dst_hbm, zeros_hbm)


def _dis(deg0_ref, deg1_ref):
    return lax.rsqrt(deg0_ref[...] + deg1_ref[...] + 1.0)


def _m1_body(x_ref, w_ref, deg0_ref, deg1_ref, o_ref):
    dis = _dis(deg0_ref, deg1_ref)
    o_ref[...] = dis * jnp.dot(x_ref[...], w_ref[...],
                               preferred_element_type=jnp.float32)


def _m2_body(p0_ref, p1_ref, deg0_ref, deg1_ref, b_ref, w_ref, o_ref):
    dis = _dis(deg0_ref, deg1_ref)
    x = jax.nn.relu(dis * (p0_ref[...] + p1_ref[...]) + b_ref[...])
    o_ref[...] = dis * jnp.dot(x, w_ref[...], preferred_element_type=jnp.float32)


def _m3_body(p0_ref, p1_ref, deg0_ref, deg1_ref, b_ref, batch_ref,
             fw1_ref, fb1_ref, fw2_ref, fb2_ref, fw3_ref, fb3_ref,
             o_ref, pooled_acc, cnt_acc):
    i = pl.program_id(0)

    @pl.when(i == 0)
    def _():
        pooled_acc[...] = jnp.zeros((128, 128), jnp.float32)
        cnt_acc[...] = jnp.zeros((128, 128), jnp.float32)

    dis = _dis(deg0_ref, deg1_ref)
    x5 = dis * (p0_ref[...] + p1_ref[...]) + b_ref[...]
    g_iota = lax.broadcasted_iota(jnp.int32, (BLK, 128), 1)
    mask = (batch_ref[...] == g_iota).astype(jnp.float32)
    dn = (((0,), (0,)), ((), ()))
    pooled_acc[...] += lax.dot_general(mask, x5, dn,
                                       preferred_element_type=jnp.float32)
    cnt_acc[...] += lax.dot_general(mask, jnp.ones((BLK, 128), jnp.float32), dn,
                                    preferred_element_type=jnp.float32)

    @pl.when(i == N_BLOCKS - 1)
    def _():
        mean = pooled_acc[...] / jnp.maximum(cnt_acc[...], 1.0)
        h1 = jax.nn.relu(jnp.dot(mean, fw1_ref[...],
                                 preferred_element_type=jnp.float32) + fb1_ref[...])
        h2 = jax.nn.relu(jnp.dot(h1, fw2_ref[...],
                                 preferred_element_type=jnp.float32) + fb2_ref[...])
        o_ref[...] = jnp.dot(h2, fw3_ref[...],
                             preferred_element_type=jnp.float32) + fb3_ref[...]


def _row_spec():
    return pl.BlockSpec((BLK, 128), lambda i: (i, 0))


def _col_spec():
    return pl.BlockSpec((BLK, 1), lambda i: (i, 0))


def _fix_spec(shape):
    return pl.BlockSpec(shape, lambda i: tuple(0 for _ in shape))


_m1 = pl.pallas_call(
    _m1_body,
    grid=(N_BLOCKS,),
    in_specs=[_row_spec(), _fix_spec((128, 128)), _col_spec(), _col_spec()],
    out_specs=_row_spec(),
    out_shape=jax.ShapeDtypeStruct((S3, 128), jnp.float32),
)

_m2 = pl.pallas_call(
    _m2_body,
    grid=(N_BLOCKS,),
    in_specs=[_row_spec(), _row_spec(), _col_spec(), _col_spec(),
              _fix_spec((1, 128)), _fix_spec((128, 128))],
    out_specs=_row_spec(),
    out_shape=jax.ShapeDtypeStruct((S3, 128), jnp.float32),
)

_m3 = pl.pallas_call(
    _m3_body,
    grid=(N_BLOCKS,),
    in_specs=[_row_spec(), _row_spec(), _col_spec(), _col_spec(),
              _fix_spec((1, 128)), _col_spec(),
              _fix_spec((128, 128)), _fix_spec((1, 128)),
              _fix_spec((128, 128)), _fix_spec((1, 128)),
              _fix_spec((128, OUT)), _fix_spec((1, OUT))],
    out_specs=_fix_spec((128, OUT)),
    out_shape=jax.ShapeDtypeStruct((128, OUT), jnp.float32),
    scratch_shapes=[pltpu.VMEM((128, 128), jnp.float32),
                    pltpu.VMEM((128, 128), jnp.float32)],
)


def kernel(anchor_x, anchor_edge_index, anchor_batch,
           positive_x, positive_edge_index, positive_batch,
           negative_x, negative_edge_index, negative_batch, params):
    f32 = jnp.float32
    i32 = jnp.int32

    xs = [anchor_x, positive_x, negative_x]
    eis = [anchor_edge_index, positive_edge_index, negative_edge_index]
    batches = [anchor_batch, positive_batch, negative_batch]

    x_stack = jnp.concatenate(
        [jnp.pad(x, ((0, N_PAD - N), (0, 0))) for x in xs])
    srcs = [jnp.pad(ei[0], (0, E_PAD - E), constant_values=N) for ei in eis]
    dsts = [jnp.pad(ei[1], (0, E_PAD - E), constant_values=N) for ei in eis]
    src_edge = jnp.stack(
        [(s + b * N_PAD).reshape(NW, CH, 128) for b, s in enumerate(srcs)])
    dst_edge = jnp.stack([d.reshape(NW, CH, 128) for d in dsts])
    deg_dst = jnp.concatenate(
        [d + b * N_PAD for b, d in enumerate(dsts)]).reshape(NW, DEG_CH, 128)
    batch_stack = jnp.concatenate(
        [jnp.pad(bt.astype(i32) + b * G, (0, N_PAD - N), constant_values=100)
         for b, bt in enumerate(batches)]).reshape(S3, 1)

    ones128 = jnp.ones((128,), f32)
    zeros1d = jnp.zeros((S3,), f32)
    zeros2d = jnp.zeros((N_PAD, 128), f32)

    degp = _deg_call(deg_dst, ones128, zeros1d)
    d0 = degp[0].reshape(S3, 1)
    d1 = degp[1].reshape(S3, 1)

    Ws = [params[f"W{i}"] for i in range(1, 6)]
    bs = [params[f"b{i}"].reshape(1, H) for i in range(1, 6)]
    fws = [params[f"fcW{i}"] for i in range(1, 4)]
    fbs = [params["fcb1"].reshape(1, 128), params["fcb2"].reshape(1, 128),
           params["fcb3"].reshape(1, OUT)]

    hp = _m1(x_stack, Ws[0], d0, d1)
    for l in range(4):
        parts = _edge_agg(hp, src_edge, dst_edge, zeros2d)
        hp = _m2(parts[0], parts[1], d0, d1, bs[l], Ws[l + 1])
    parts = _edge_agg(hp, src_edge, dst_edge, zeros2d)
    o = _m3(parts[0], parts[1], d0, d1, bs[4], batch_stack,
            fws[0], fbs[0], fws[1], fbs[1], fws[2], fbs[2])

    return (o[0:G, :], o[G:2 * G, :], o[2 * G:3 * G, :])
